# baseline (speedup 1.0000x reference)
import jax
import jax.numpy as jnp
from jax import lax
from jax.experimental import pallas as pl
from jax.experimental.pallas import tpu as pltpu
from jax.experimental.pallas import tpu_sc as plsc

_E = 16
_BT = 2048
_NC = 2
_NS = 16
_NW = _NC * _NS
_L = 16


def _logits_block(x_ref, w1_ref, b1_ref, w2_ref, b2_ref, logits_ref):
    h = jnp.maximum(
        jnp.dot(x_ref[...], w1_ref[...],
                preferred_element_type=jnp.float32) + b1_ref[...],
        0.0)
    logits_ref[...] = jnp.dot(
        h, w2_ref[...], preferred_element_type=jnp.float32) + b2_ref[...]


def _tc_logits(x, W1, b1r, W2, b2r):
    tokens, input_dim = x.shape
    hidden = W1.shape[1]
    return pl.pallas_call(
        _logits_block,
        grid=(tokens // _BT,),
        in_specs=[
            pl.BlockSpec((_BT, input_dim), lambda i: (i, 0)),
            pl.BlockSpec((input_dim, hidden), lambda i: (0, 0)),
            pl.BlockSpec((1, hidden), lambda i: (0, 0)),
            pl.BlockSpec((hidden, _E), lambda i: (0, 0)),
            pl.BlockSpec((1, _E), lambda i: (0, 0)),
        ],
        out_specs=pl.BlockSpec((_BT, _E), lambda i: (i, 0)),
        out_shape=jax.ShapeDtypeStruct((tokens, _E), jnp.float32),
    )(x, W1, b1r, W2, b2r)


def _sc_routing_body(logits_hbm, ori_hbm, router_hbm, idx_hbm,
                     lg_v, rt_v, idx_v):
    n = lg_v.shape[0]
    rows_per_w = n // _E
    wid = lax.axis_index("s") * _NC + lax.axis_index("c")
    base = pl.multiple_of(wid * n, n)
    pltpu.sync_copy(logits_hbm.at[pl.ds(base, n)], lg_v)

    lane = lax.broadcasted_iota(jnp.int32, (_L,), 0)
    neg_inf = jnp.full((_L,), -jnp.inf, dtype=jnp.float32)
    zero = jnp.zeros((_L,), dtype=jnp.float32)
    cols = [jnp.full((_L,), e, dtype=jnp.int32) for e in range(_E)]

    @plsc.parallel_loop(0, rows_per_w // _L, unroll=4)
    def tile_body(t):
        rows = t * _L + lane
        rx16 = rows * _E
        gidx = [rx16 + e for e in range(_E)]
        g = [plsc.load_gather(lg_v, (gidx[e],)) for e in range(_E)]

        m1 = g[0]
        for e in range(1, _E):
            m1 = jnp.maximum(m1, g[e])
        i1 = jnp.full((_L,), _E, dtype=jnp.int32)
        for e in range(_E - 1, -1, -1):
            i1 = jnp.where(g[e] == m1, cols[e], i1)
        gm = [jnp.where(i1 == e, neg_inf, g[e]) for e in range(_E)]
        m2 = gm[0]
        for e in range(1, _E):
            m2 = jnp.maximum(m2, gm[e])
        i2 = jnp.full((_L,), _E, dtype=jnp.int32)
        for e in range(_E - 1, -1, -1):
            i2 = jnp.where(gm[e] == m2, cols[e], i2)

        rx2 = rows * 2
        plsc.store_scatter(idx_v, (rx2,), i1)
        plsc.store_scatter(idx_v, (rx2 + 1,), i2)

        ex = [jnp.exp((g[e] - m1) * 100.0) for e in range(_E)]
        s = ex[0]
        for e in range(1, _E):
            s = s + ex[e]
        inv = 1.0 / s
        for e in range(_E):
            plsc.store_scatter(lg_v, (gidx[e],), ex[e] * inv)

        for j in range(_L):
            off = pl.multiple_of(t * (_L * _E) + j * _L, _L)
            rt_v[pl.ds(off, _L)] = zero
        p2 = jnp.exp(m2 - m1)
        den = 1.0 + p2
        plsc.store_scatter(rt_v, (rx16 + i1,), 1.0 / den)
        plsc.store_scatter(rt_v, (rx16 + i2,), p2 / den)

    pltpu.sync_copy(lg_v, ori_hbm.at[pl.ds(base, n)])
    pltpu.sync_copy(rt_v, router_hbm.at[pl.ds(base, n)])
    base2 = pl.multiple_of(wid * (n * 2 // _E), n * 2 // _E)
    pltpu.sync_copy(idx_v, idx_hbm.at[pl.ds(base2, n * 2 // _E)])


def _sc_routing(logits):
    tokens = logits.shape[0]
    rows_per_w = tokens // _NW
    n = rows_per_w * _E
    mesh = plsc.VectorSubcoreMesh(
        core_axis_name="c", subcore_axis_name="s",
        num_cores=_NC, num_subcores=_NS)
    f = pl.kernel(
        _sc_routing_body,
        out_type=(
            jax.ShapeDtypeStruct((tokens * _E,), jnp.float32),
            jax.ShapeDtypeStruct((tokens * _E,), jnp.float32),
            jax.ShapeDtypeStruct((tokens * 2,), jnp.int32),
        ),
        mesh=mesh,
        compiler_params=pltpu.CompilerParams(needs_layout_passes=False),
        scratch_types=[
            pltpu.VMEM((n,), jnp.float32),
            pltpu.VMEM((n,), jnp.float32),
            pltpu.VMEM((rows_per_w * 2,), jnp.int32),
        ],
    )
    ori_f, rt_f, idx_f = f(logits.reshape(-1))
    return (ori_f.reshape(tokens, _E), rt_f.reshape(tokens, _E),
            idx_f.reshape(tokens, 2))


def kernel(x, W1, b1, W2, b2):
    hidden = W1.shape[1]
    b1r = b1.reshape(1, hidden)
    b2r = b2.reshape(1, _E)
    logits = _tc_logits(x, W1, b1r, W2, b2r)
    ori, router, idx = _sc_routing(logits)
    return (ori, router, idx)


# speedup vs baseline: 1.0335x; 1.0335x over previous
import jax
import jax.numpy as jnp
from jax import lax
from jax.experimental import pallas as pl
from jax.experimental.pallas import tpu as pltpu
from jax.experimental.pallas import tpu_sc as plsc

_E = 16
_BT = 2048
_NC = 2
_NS = 16
_NW = _NC * _NS
_L = 16


def _logits_block(x_ref, w1_ref, b1_ref, w2_ref, b2_ref, logits_ref, ori_ref):
    h = jnp.maximum(
        jnp.dot(x_ref[...], w1_ref[...],
                preferred_element_type=jnp.float32) + b1_ref[...],
        0.0)
    logits = jnp.dot(
        h, w2_ref[...], preferred_element_type=jnp.float32) + b2_ref[...]
    logits_ref[...] = logits
    m1 = jnp.max(logits, axis=1, keepdims=True)
    e = jnp.exp((logits - m1) * 100.0)
    ori_ref[...] = e / jnp.sum(e, axis=1, keepdims=True)


def _tc_logits(x, W1, b1r, W2, b2r):
    tokens, input_dim = x.shape
    hidden = W1.shape[1]
    return pl.pallas_call(
        _logits_block,
        grid=(tokens // _BT,),
        in_specs=[
            pl.BlockSpec((_BT, input_dim), lambda i: (i, 0)),
            pl.BlockSpec((input_dim, hidden), lambda i: (0, 0)),
            pl.BlockSpec((1, hidden), lambda i: (0, 0)),
            pl.BlockSpec((hidden, _E), lambda i: (0, 0)),
            pl.BlockSpec((1, _E), lambda i: (0, 0)),
        ],
        out_specs=(pl.BlockSpec((_BT, _E), lambda i: (i, 0)),
                   pl.BlockSpec((_BT, _E), lambda i: (i, 0))),
        out_shape=(jax.ShapeDtypeStruct((tokens, _E), jnp.float32),
                   jax.ShapeDtypeStruct((tokens, _E), jnp.float32)),
    )(x, W1, b1r, W2, b2r)


def _sc_routing_body(logits_hbm, router_hbm, idx_hbm,
                     lg_v, rt_v, idx_v):
    n = lg_v.shape[0]
    rows_per_w = n // _E
    wid = lax.axis_index("s") * _NC + lax.axis_index("c")
    base = pl.multiple_of(wid * n, n)
    pltpu.sync_copy(logits_hbm.at[pl.ds(base, n)], lg_v)

    lane = lax.broadcasted_iota(jnp.int32, (_L,), 0)
    neg_inf = jnp.full((_L,), -jnp.inf, dtype=jnp.float32)
    zero = jnp.zeros((_L,), dtype=jnp.float32)
    cols = [jnp.full((_L,), e, dtype=jnp.int32) for e in range(_E)]

    def tile_body(t, carry):
        rows = t * _L + lane
        rx16 = rows * _E
        gidx = [rx16 + e for e in range(_E)]
        g = [plsc.load_gather(lg_v, (gidx[e],)) for e in range(_E)]

        m1 = g[0]
        for e in range(1, _E):
            m1 = jnp.maximum(m1, g[e])
        i1 = jnp.full((_L,), _E, dtype=jnp.int32)
        for e in range(_E - 1, -1, -1):
            i1 = jnp.where(g[e] == m1, cols[e], i1)
        gm = [jnp.where(i1 == e, neg_inf, g[e]) for e in range(_E)]
        m2 = gm[0]
        for e in range(1, _E):
            m2 = jnp.maximum(m2, gm[e])
        i2 = jnp.full((_L,), _E, dtype=jnp.int32)
        for e in range(_E - 1, -1, -1):
            i2 = jnp.where(gm[e] == m2, cols[e], i2)

        rx2 = rows * 2
        plsc.store_scatter(idx_v, (rx2,), i1)
        plsc.store_scatter(idx_v, (rx2 + 1,), i2)

        for j in range(_L):
            off = pl.multiple_of(t * (_L * _E) + j * _L, _L)
            rt_v[pl.ds(off, _L)] = zero
        p2 = jnp.exp(m2 - m1)
        den = 1.0 + p2
        plsc.store_scatter(rt_v, (rx16 + i1,), 1.0 / den)
        plsc.store_scatter(rt_v, (rx16 + i2,), p2 / den)
        return carry

    lax.fori_loop(0, rows_per_w // _L, tile_body, 0)

    pltpu.sync_copy(rt_v, router_hbm.at[pl.ds(base, n)])
    base2 = pl.multiple_of(wid * (n * 2 // _E), n * 2 // _E)
    pltpu.sync_copy(idx_v, idx_hbm.at[pl.ds(base2, n * 2 // _E)])


def _sc_routing(logits):
    tokens = logits.shape[0]
    rows_per_w = tokens // _NW
    n = rows_per_w * _E
    mesh = plsc.VectorSubcoreMesh(
        core_axis_name="c", subcore_axis_name="s",
        num_cores=_NC, num_subcores=_NS)
    f = pl.kernel(
        _sc_routing_body,
        out_type=(
            jax.ShapeDtypeStruct((tokens * _E,), jnp.float32),
            jax.ShapeDtypeStruct((tokens * 2,), jnp.int32),
        ),
        mesh=mesh,
        compiler_params=pltpu.CompilerParams(needs_layout_passes=False),
        scratch_types=[
            pltpu.VMEM((n,), jnp.float32),
            pltpu.VMEM((n,), jnp.float32),
            pltpu.VMEM((rows_per_w * 2,), jnp.int32),
        ],
    )
    rt_f, idx_f = f(logits.reshape(-1))
    return (rt_f.reshape(tokens, _E), idx_f.reshape(tokens, 2))


def kernel(x, W1, b1, W2, b2):
    hidden = W1.shape[1]
    b1r = b1.reshape(1, hidden)
    b2r = b2.reshape(1, _E)
    logits, ori = _tc_logits(x, W1, b1r, W2, b2r)
    router, idx = _sc_routing(logits)
    return (ori, router, idx)


# speedup vs baseline: 1.0342x; 1.0006x over previous
import jax
import jax.numpy as jnp
from jax import lax
from jax.experimental import pallas as pl
from jax.experimental.pallas import tpu as pltpu
from jax.experimental.pallas import tpu_sc as plsc

_E = 16
_BT = 2048
_NC = 2
_NS = 16
_NW = _NC * _NS
_L = 16


def _logits_block(x_ref, w1_ref, b1_ref, w2_ref, b2_ref, logits_ref, ori_ref):
    h = jnp.maximum(
        jnp.dot(x_ref[...], w1_ref[...],
                preferred_element_type=jnp.float32) + b1_ref[...],
        0.0)
    logits = jnp.dot(
        h, w2_ref[...], preferred_element_type=jnp.float32) + b2_ref[...]
    logits_ref[...] = logits
    m1 = jnp.max(logits, axis=1, keepdims=True)
    e = jnp.exp((logits - m1) * 100.0)
    ori_ref[...] = e / jnp.sum(e, axis=1, keepdims=True)


def _tc_logits(x, W1, b1r, W2, b2r):
    tokens, input_dim = x.shape
    hidden = W1.shape[1]
    return pl.pallas_call(
        _logits_block,
        grid=(tokens // _BT,),
        in_specs=[
            pl.BlockSpec((_BT, input_dim), lambda i: (i, 0)),
            pl.BlockSpec((input_dim, hidden), lambda i: (0, 0)),
            pl.BlockSpec((1, hidden), lambda i: (0, 0)),
            pl.BlockSpec((hidden, _E), lambda i: (0, 0)),
            pl.BlockSpec((1, _E), lambda i: (0, 0)),
        ],
        out_specs=(pl.BlockSpec((_BT, _E), lambda i: (i, 0)),
                   pl.BlockSpec((_BT, _E), lambda i: (i, 0))),
        out_shape=(jax.ShapeDtypeStruct((tokens, _E), jnp.float32),
                   jax.ShapeDtypeStruct((tokens, _E), jnp.float32)),
    )(x, W1, b1r, W2, b2r)


def _sc_routing_body(logits_hbm, router_hbm, idx_hbm,
                     lg_v, rt_v, idx_v):
    n = lg_v.shape[0]
    rows_per_w = n // _E
    wid = lax.axis_index("s") * _NC + lax.axis_index("c")
    base = pl.multiple_of(wid * n, n)
    pltpu.sync_copy(logits_hbm.at[pl.ds(base, n)], lg_v)

    lane = lax.broadcasted_iota(jnp.int32, (_L,), 0)
    neg_inf = jnp.full((_L,), -jnp.inf, dtype=jnp.float32)
    zero = jnp.zeros((_L,), dtype=jnp.float32)
    cols = [jnp.full((_L,), e, dtype=jnp.int32) for e in range(_E)]

    def tile_body(t, carry):
        rows = t * _L + lane
        rx16 = rows * _E
        gidx = [rx16 + e for e in range(_E)]
        g = [plsc.load_gather(lg_v, (gidx[e],)) for e in range(_E)]

        m1 = g[0]
        for e in range(1, _E):
            m1 = jnp.maximum(m1, g[e])
        i1 = jnp.full((_L,), _E, dtype=jnp.int32)
        for e in range(_E - 1, -1, -1):
            i1 = jnp.where(g[e] == m1, cols[e], i1)
        gm = [jnp.where(i1 == e, neg_inf, g[e]) for e in range(_E)]
        m2 = gm[0]
        for e in range(1, _E):
            m2 = jnp.maximum(m2, gm[e])
        i2 = jnp.full((_L,), _E, dtype=jnp.int32)
        for e in range(_E - 1, -1, -1):
            i2 = jnp.where(gm[e] == m2, cols[e], i2)

        rx2 = rows * 2
        plsc.store_scatter(idx_v, (rx2,), i1)
        plsc.store_scatter(idx_v, (rx2 + 1,), i2)

        for j in range(_L):
            off = pl.multiple_of(t * (_L * _E) + j * _L, _L)
            rt_v[pl.ds(off, _L)] = zero
        p2 = jnp.exp(m2 - m1)
        den = 1.0 + p2
        plsc.store_scatter(rt_v, (rx16 + i1,), 1.0 / den)
        plsc.store_scatter(rt_v, (rx16 + i2,), p2 / den)
        return carry

    lax.fori_loop(0, rows_per_w // _L, tile_body, 0)

    pltpu.sync_copy(rt_v, router_hbm.at[pl.ds(base, n)])
    base2 = pl.multiple_of(wid * (n * 2 // _E), n * 2 // _E)
    pltpu.sync_copy(idx_v, idx_hbm.at[pl.ds(base2, n * 2 // _E)])


def _sc_routing(logits):
    tokens = logits.shape[0]
    rows_per_w = tokens // _NW
    n = rows_per_w * _E
    mesh = plsc.VectorSubcoreMesh(
        core_axis_name="c", subcore_axis_name="s",
        num_cores=_NC, num_subcores=_NS)
    f = pl.kernel(
        _sc_routing_body,
        out_type=(
            jax.ShapeDtypeStruct((tokens * _E,), jnp.float32),
            jax.ShapeDtypeStruct((tokens * 2,), jnp.int32),
        ),
        mesh=mesh,
        compiler_params=pltpu.CompilerParams(
            needs_layout_passes=False,
            disable_bounds_checks=True,
            disable_semaphore_checks=True),
        scratch_types=[
            pltpu.VMEM((n,), jnp.float32),
            pltpu.VMEM((n,), jnp.float32),
            pltpu.VMEM((rows_per_w * 2,), jnp.int32),
        ],
    )
    rt_f, idx_f = f(logits.reshape(-1))
    return (rt_f.reshape(tokens, _E), idx_f.reshape(tokens, 2))


def kernel(x, W1, b1, W2, b2):
    hidden = W1.shape[1]
    b1r = b1.reshape(1, hidden)
    b2r = b2.reshape(1, _E)
    logits, ori = _tc_logits(x, W1, b1r, W2, b2r)
    router, idx = _sc_routing(logits)
    return (ori, router, idx)
